# baseline (speedup 1.0000x reference)
import functools

import jax
import jax.numpy as jnp
from jax import lax
from jax.experimental import pallas as pl
from jax.experimental.pallas import tpu as pltpu
from jax.experimental.pallas import tpu_sc as plsc

N = 50000
D = 512
B = 64
NW = 32
PER = 1568
BBLK = 1024
NB = -(-N // BBLK)
NEG = -3.0e38

_mesh = plsc.VectorSubcoreMesh(core_axis_name="c", subcore_axis_name="s")
_sc_params = pltpu.CompilerParams(use_tc_tiling_on_sc=True,
                                  needs_layout_passes=False)


def _insert3(cs, ci, t0s, t1s, t2s, t0i, t1i, t2i):
    gt0 = cs > t0s
    gt1 = cs > t1s
    gt2 = cs > t2s
    n0s = jnp.where(gt0, cs, t0s)
    n0i = jnp.where(gt0, ci, t0i)
    n1s = jnp.where(gt0, t0s, jnp.where(gt1, cs, t1s))
    n1i = jnp.where(gt0, t0i, jnp.where(gt1, ci, t1i))
    n2s = jnp.where(gt1, t1s, jnp.where(gt2, cs, t2s))
    n2i = jnp.where(gt1, t1i, jnp.where(gt2, ci, t2i))
    return n0s, n1s, n2s, n0i, n1i, n2i


def _ktc_body(x_ref, b_ref, psum_ref, pcnt_ref, sc_ref):
    i = pl.program_id(0)
    xb = x_ref[...]
    bb = b_ref[...]
    rowmask = (lax.broadcasted_iota(jnp.int32, (BBLK, D), 0)
               + i * BBLK) < N
    xb = jnp.where(rowmask, xb, jnp.float32(0.0))
    seg = lax.broadcasted_iota(jnp.int32, (B, BBLK), 0)
    gcol = lax.broadcasted_iota(jnp.int32, (B, BBLK), 1) + i * BBLK
    onehot = jnp.where((seg == bb[None, :]) & (gcol < N), jnp.float32(1.0),
                       jnp.float32(0.0))
    ps = jnp.dot(onehot, xb, preferred_element_type=jnp.float32,
                 precision=lax.Precision.HIGHEST)
    cnt = jnp.broadcast_to(jnp.sum(onehot, axis=1, keepdims=True), (B, 128))

    @pl.when(i == 0)
    def _():
        psum_ref[...] = ps
        pcnt_ref[...] = cnt

    @pl.when(i > 0)
    def _():
        psum_ref[...] = psum_ref[...] + ps
        pcnt_ref[...] = pcnt_ref[...] + cnt

    sc_ref[...] = xb[:, D - 1]


_ktc = pl.pallas_call(
    _ktc_body,
    grid=(NB,),
    in_specs=[
        pl.BlockSpec((BBLK, D), lambda i: (i, 0)),
        pl.BlockSpec((BBLK,), lambda i: (i,)),
    ],
    out_specs=[
        pl.BlockSpec((B, D), lambda i: (0, 0)),
        pl.BlockSpec((B, 128), lambda i: (0, 0)),
        pl.BlockSpec((BBLK,), lambda i: (i,)),
    ],
    out_shape=[
        jax.ShapeDtypeStruct((B, D), jnp.float32),
        jax.ShapeDtypeStruct((B, 128), jnp.float32),
        jax.ShapeDtypeStruct((N,), jnp.float32),
    ],
)


@functools.partial(
    pl.kernel,
    out_type=(
        jax.ShapeDtypeStruct((NW, B, 16), jnp.float32),
        jax.ShapeDtypeStruct((NW, B, 16), jnp.int32),
    ),
    mesh=_mesh,
    compiler_params=_sc_params,
    scratch_types=[
        pltpu.VMEM((PER,), jnp.float32),
        pltpu.VMEM((PER,), jnp.int32),
        pltpu.VMEM((B, 16), jnp.float32),
        pltpu.VMEM((B, 16), jnp.int32),
    ],
)
def _ksc1(sc_hbm, bat_hbm, cands_hbm, candi_hbm, sc_v, idx_v, ts_s, ts_i):
    cid = lax.axis_index("c")
    sid = lax.axis_index("s")
    wid = cid * 16 + sid
    start = wid * PER
    sstart = jnp.minimum(start, N - PER)
    iota = lax.iota(jnp.int32, 16)

    def init_body(r, _):
        ts_s[r, :] = jnp.full((16,), NEG, jnp.float32)
        ts_i[r, :] = jnp.zeros((16,), jnp.int32)
        return 0

    lax.fori_loop(0, B, init_body, 0)
    pltpu.sync_copy(sc_hbm.at[pl.ds(sstart, PER)], sc_v)
    pltpu.sync_copy(bat_hbm.at[pl.ds(sstart, PER)], idx_v)

    def grp(kk, _):
        goff = pl.multiple_of(kk * 16, 16)
        bv = idx_v[pl.ds(goff, 16)]
        svec = sc_v[pl.ds(goff, 16)]
        gbase = sstart + kk * 16
        gvec = jnp.broadcast_to(gbase, (16,)) + iota
        s_eff = jnp.where(gvec >= start, svec, jnp.float32(NEG))
        thr = plsc.load_gather(ts_s, [bv, jnp.full((16,), 2, jnp.int32)])
        npass = plsc.all_reduce_population_count(s_eff > thr)[0]

        @pl.when(npass > 0)
        def _():
            for l in range(16):
                se = s_eff[l]
                b = bv[l]
                g = gbase + l
                sv = ts_s[b, :]
                iv = ts_i[b, :]
                n0s, n1s, n2s, n0i, n1i, n2i = _insert3(
                    se, g, sv[0], sv[1], sv[2], iv[0], iv[1], iv[2])
                ns = jnp.where(iota == 0, n0s,
                               jnp.where(iota == 1, n1s,
                                         jnp.where(iota == 2, n2s, sv)))
                ni = jnp.where(iota == 0, n0i,
                               jnp.where(iota == 1, n1i,
                                         jnp.where(iota == 2, n2i, iv)))
                ts_s[b, :] = ns
                ts_i[b, :] = ni

        return 0

    lax.fori_loop(0, PER // 16, grp, 0)
    pltpu.sync_copy(ts_s, cands_hbm.at[wid])
    pltpu.sync_copy(ts_i, candi_hbm.at[wid])


@functools.partial(
    pl.kernel,
    out_type=jax.ShapeDtypeStruct((B, 5 * D), jnp.float32),
    mesh=_mesh,
    compiler_params=_sc_params,
    scratch_types=[
        pltpu.VMEM((NW, 1, 16), jnp.float32),
        pltpu.VMEM((NW, 1, 16), jnp.int32),
        pltpu.VMEM((1, D), jnp.float32),
        pltpu.VMEM((1, 128), jnp.float32),
        pltpu.VMEM((NW, 1, 16), jnp.float32),
        pltpu.VMEM((NW, 1, 16), jnp.int32),
        pltpu.VMEM((1, D), jnp.float32),
        pltpu.VMEM((1, 128), jnp.float32),
        pltpu.VMEM((16,), jnp.int32),
        pltpu.VMEM((16, D), jnp.float32),
        pltpu.VMEM((5 * D,), jnp.float32),
        pltpu.VMEM((5 * D,), jnp.float32),
        pltpu.SemaphoreType.DMA,
        pltpu.SemaphoreType.DMA,
        pltpu.SemaphoreType.DMA,
        pltpu.SemaphoreType.DMA,
    ],
)
def _ksc2(x_hbm, psum_hbm, pcnt_hbm, cands_hbm, candi_hbm, out_hbm,
          cs0, ci0, ps0, pc0, cs1, ci1, ps1, pc1, gi_v, grows_v,
          orow0, orow1, sem_in0, sem_in1, sem_g, sem_out):
    cid = lax.axis_index("c")
    sid = lax.axis_index("s")
    wid = cid * 16 + sid
    iota = lax.iota(jnp.int32, 16)
    zeros16 = jnp.zeros((16,), jnp.float32)

    def in_copies(seg, cs_v, ci_v, psv, pcv, sem):
        return (
            pltpu.make_async_copy(
                cands_hbm.at[:, pl.ds(seg, 1), :], cs_v, sem),
            pltpu.make_async_copy(
                candi_hbm.at[:, pl.ds(seg, 1), :], ci_v, sem),
            pltpu.make_async_copy(
                psum_hbm.at[pl.ds(seg, 1), :], psv, sem),
            pltpu.make_async_copy(
                pcnt_hbm.at[pl.ds(seg, 1), :], pcv, sem),
        )

    def do_seg(seg, cs_v, ci_v, psv, pcv, orow_v, sem):
        for c in in_copies(seg, cs_v, ci_v, psv, pcv, sem):
            c.wait()

        def m_body(t, carry):
            t0s, t1s, t2s, t0i, t1i, t2i = carry
            csv = cs_v[t, 0, :]
            civ = ci_v[t, 0, :]
            for k in range(3):
                t0s, t1s, t2s, t0i, t1i, t2i = _insert3(
                    csv[k], civ[k], t0s, t1s, t2s, t0i, t1i, t2i)
            return (t0s, t1s, t2s, t0i, t1i, t2i)

        z = jnp.int32(0)
        ng = jnp.float32(NEG)
        t0s, t1s, t2s, t0i, t1i, t2i = lax.fori_loop(
            0, NW, m_body, (ng, ng, ng, z, z, z))

        cnt = pcv[0, pl.ds(0, 16)]
        cntc = jnp.maximum(cnt, jnp.float32(1.0))
        one = jnp.full((16,), 1.0, jnp.float32)
        v0 = jnp.where(cnt > 0.5, one, zeros16)
        v1 = jnp.where(cnt > 1.5, one, zeros16)
        v2 = jnp.where(cnt > 2.5, one, zeros16)

        gi_v[...] = jnp.where(iota == 0, t0i,
                              jnp.where(iota == 1, t1i,
                                        jnp.where(iota == 2, t2i, z)))
        pltpu.async_copy(x_hbm.at[gi_v], grows_v, sem_g).wait()

        def col_body(c4, _):
            bases = [pl.multiple_of(c4 * 64 + u * 16, 16) for u in range(4)]
            for u in range(4):
                base = bases[u]
                sl = pl.ds(base, 16)
                sv = psv[0, sl]
                orow_v[pl.ds(base, 16)] = sv / cntc
                orow_v[pl.ds(D + base, 16)] = sv
                orow_v[pl.ds(2 * D + base, 16)] = grows_v[0, sl] * v0
                orow_v[pl.ds(3 * D + base, 16)] = grows_v[1, sl] * v1
                orow_v[pl.ds(4 * D + base, 16)] = grows_v[2, sl] * v2
            return 0

        lax.fori_loop(0, D // 64, col_body, 0)
        pltpu.async_copy(orow_v, out_hbm.at[seg], sem_out)

    seg_a = wid * 2
    seg_b = seg_a + 1
    for c in in_copies(seg_a, cs0, ci0, ps0, pc0, sem_in0):
        c.start()
    for c in in_copies(seg_b, cs1, ci1, ps1, pc1, sem_in1):
        c.start()
    do_seg(seg_a, cs0, ci0, ps0, pc0, orow0, sem_in0)
    do_seg(seg_b, cs1, ci1, ps1, pc1, orow1, sem_in1)
    pltpu.make_async_copy(orow0, out_hbm.at[seg_a], sem_out).wait()
    pltpu.make_async_copy(orow1, out_hbm.at[seg_b], sem_out).wait()


def kernel(x, batch):
    bat = batch.astype(jnp.int32)
    psum, pcnt, scores = _ktc(x, bat)
    cs, ci = _ksc1(scores, bat)
    return _ksc2(x, psum, pcnt, cs, ci)


# speedup vs baseline: 1.1651x; 1.1651x over previous
import functools

import jax
import jax.numpy as jnp
from jax import lax
from jax.experimental import pallas as pl
from jax.experimental.pallas import tpu as pltpu
from jax.experimental.pallas import tpu_sc as plsc

N = 50000
D = 512
B = 64
NW = 32
PER = 1568
BBLK = 1024
NB = -(-N // BBLK)
NEG = -3.0e38

_mesh = plsc.VectorSubcoreMesh(core_axis_name="c", subcore_axis_name="s")
_sc_params = pltpu.CompilerParams(use_tc_tiling_on_sc=True,
                                  needs_layout_passes=False)


def _insert3(cs, ci, t0s, t1s, t2s, t0i, t1i, t2i):
    gt0 = cs > t0s
    gt1 = cs > t1s
    gt2 = cs > t2s
    n0s = jnp.where(gt0, cs, t0s)
    n0i = jnp.where(gt0, ci, t0i)
    n1s = jnp.where(gt0, t0s, jnp.where(gt1, cs, t1s))
    n1i = jnp.where(gt0, t0i, jnp.where(gt1, ci, t1i))
    n2s = jnp.where(gt1, t1s, jnp.where(gt2, cs, t2s))
    n2i = jnp.where(gt1, t1i, jnp.where(gt2, ci, t2i))
    return n0s, n1s, n2s, n0i, n1i, n2i


def _ktc_body(x_ref, b_ref, psum_ref, pcnt_ref, sc_ref):
    i = pl.program_id(0)
    xb = x_ref[...]
    bb = b_ref[...]
    rowmask = (lax.broadcasted_iota(jnp.int32, (BBLK, D), 0)
               + i * BBLK) < N
    xb = jnp.where(rowmask, xb, jnp.float32(0.0))
    seg = lax.broadcasted_iota(jnp.int32, (B, BBLK), 0)
    gcol = lax.broadcasted_iota(jnp.int32, (B, BBLK), 1) + i * BBLK
    onehot = jnp.where((seg == bb[None, :]) & (gcol < N), jnp.float32(1.0),
                       jnp.float32(0.0))
    hi = xb.astype(jnp.bfloat16).astype(jnp.float32)
    lo = xb - hi
    ps = (jnp.dot(onehot, hi, preferred_element_type=jnp.float32)
          + jnp.dot(onehot, lo, preferred_element_type=jnp.float32))
    cnt = jnp.broadcast_to(jnp.sum(onehot, axis=1, keepdims=True), (B, 128))

    @pl.when(i == 0)
    def _():
        psum_ref[...] = ps
        pcnt_ref[...] = cnt

    @pl.when(i > 0)
    def _():
        psum_ref[...] = psum_ref[...] + ps
        pcnt_ref[...] = pcnt_ref[...] + cnt

    sc_ref[...] = xb[:, D - 1]


_ktc = pl.pallas_call(
    _ktc_body,
    grid=(NB,),
    in_specs=[
        pl.BlockSpec((BBLK, D), lambda i: (i, 0)),
        pl.BlockSpec((BBLK,), lambda i: (i,)),
    ],
    out_specs=[
        pl.BlockSpec((B, D), lambda i: (0, 0)),
        pl.BlockSpec((B, 128), lambda i: (0, 0)),
        pl.BlockSpec((BBLK,), lambda i: (i,)),
    ],
    out_shape=[
        jax.ShapeDtypeStruct((B, D), jnp.float32),
        jax.ShapeDtypeStruct((B, 128), jnp.float32),
        jax.ShapeDtypeStruct((N,), jnp.float32),
    ],
)


@functools.partial(
    pl.kernel,
    out_type=(
        jax.ShapeDtypeStruct((NW, B, 16), jnp.float32),
        jax.ShapeDtypeStruct((NW, B, 16), jnp.int32),
    ),
    mesh=_mesh,
    compiler_params=_sc_params,
    scratch_types=[
        pltpu.VMEM((PER,), jnp.float32),
        pltpu.VMEM((PER,), jnp.int32),
        pltpu.VMEM((B, 16), jnp.float32),
        pltpu.VMEM((B, 16), jnp.int32),
    ],
)
def _ksc1(sc_hbm, bat_hbm, cands_hbm, candi_hbm, sc_v, idx_v, ts_s, ts_i):
    cid = lax.axis_index("c")
    sid = lax.axis_index("s")
    wid = cid * 16 + sid
    start = wid * PER
    sstart = jnp.minimum(start, N - PER)
    iota = lax.iota(jnp.int32, 16)

    def init_body(r, _):
        ts_s[r, :] = jnp.full((16,), NEG, jnp.float32)
        ts_i[r, :] = jnp.zeros((16,), jnp.int32)
        return 0

    lax.fori_loop(0, B, init_body, 0)
    pltpu.sync_copy(sc_hbm.at[pl.ds(sstart, PER)], sc_v)
    pltpu.sync_copy(bat_hbm.at[pl.ds(sstart, PER)], idx_v)

    def grp(kk, _):
        goff = pl.multiple_of(kk * 16, 16)
        bv = idx_v[pl.ds(goff, 16)]
        svec = sc_v[pl.ds(goff, 16)]
        gbase = sstart + kk * 16
        gvec = jnp.broadcast_to(gbase, (16,)) + iota
        s_eff = jnp.where(gvec >= start, svec, jnp.float32(NEG))
        thr = plsc.load_gather(ts_s, [bv, jnp.full((16,), 2, jnp.int32)])
        npass = plsc.all_reduce_population_count(s_eff > thr)[0]

        @pl.when(npass > 0)
        def _():
            for l in range(16):
                se = s_eff[l]
                b = bv[l]
                g = gbase + l
                sv = ts_s[b, :]
                iv = ts_i[b, :]
                n0s, n1s, n2s, n0i, n1i, n2i = _insert3(
                    se, g, sv[0], sv[1], sv[2], iv[0], iv[1], iv[2])
                ns = jnp.where(iota == 0, n0s,
                               jnp.where(iota == 1, n1s,
                                         jnp.where(iota == 2, n2s, sv)))
                ni = jnp.where(iota == 0, n0i,
                               jnp.where(iota == 1, n1i,
                                         jnp.where(iota == 2, n2i, iv)))
                ts_s[b, :] = ns
                ts_i[b, :] = ni

        return 0

    lax.fori_loop(0, PER // 16, grp, 0)
    pltpu.sync_copy(ts_s, cands_hbm.at[wid])
    pltpu.sync_copy(ts_i, candi_hbm.at[wid])


@functools.partial(
    pl.kernel,
    out_type=jax.ShapeDtypeStruct((B, 5 * D), jnp.float32),
    mesh=_mesh,
    compiler_params=_sc_params,
    scratch_types=[
        pltpu.VMEM((NW, 1, 16), jnp.float32),
        pltpu.VMEM((NW, 1, 16), jnp.int32),
        pltpu.VMEM((1, D), jnp.float32),
        pltpu.VMEM((1, 128), jnp.float32),
        pltpu.VMEM((NW, 1, 16), jnp.float32),
        pltpu.VMEM((NW, 1, 16), jnp.int32),
        pltpu.VMEM((1, D), jnp.float32),
        pltpu.VMEM((1, 128), jnp.float32),
        pltpu.VMEM((16,), jnp.int32),
        pltpu.VMEM((16, D), jnp.float32),
        pltpu.VMEM((5 * D,), jnp.float32),
        pltpu.VMEM((5 * D,), jnp.float32),
        pltpu.SemaphoreType.DMA,
        pltpu.SemaphoreType.DMA,
        pltpu.SemaphoreType.DMA,
        pltpu.SemaphoreType.DMA,
    ],
)
def _ksc2(x_hbm, psum_hbm, pcnt_hbm, cands_hbm, candi_hbm, out_hbm,
          cs0, ci0, ps0, pc0, cs1, ci1, ps1, pc1, gi_v, grows_v,
          orow0, orow1, sem_in0, sem_in1, sem_g, sem_out):
    cid = lax.axis_index("c")
    sid = lax.axis_index("s")
    wid = cid * 16 + sid
    iota = lax.iota(jnp.int32, 16)
    zeros16 = jnp.zeros((16,), jnp.float32)

    def in_copies(seg, cs_v, ci_v, psv, pcv, sem):
        return (
            pltpu.make_async_copy(
                cands_hbm.at[:, pl.ds(seg, 1), :], cs_v, sem),
            pltpu.make_async_copy(
                candi_hbm.at[:, pl.ds(seg, 1), :], ci_v, sem),
            pltpu.make_async_copy(
                psum_hbm.at[pl.ds(seg, 1), :], psv, sem),
            pltpu.make_async_copy(
                pcnt_hbm.at[pl.ds(seg, 1), :], pcv, sem),
        )

    def do_seg(seg, cs_v, ci_v, psv, pcv, orow_v, sem):
        for c in in_copies(seg, cs_v, ci_v, psv, pcv, sem):
            c.wait()

        def m_body(t, carry):
            t0s, t1s, t2s, t0i, t1i, t2i = carry
            csv = cs_v[t, 0, :]
            civ = ci_v[t, 0, :]
            for k in range(3):
                t0s, t1s, t2s, t0i, t1i, t2i = _insert3(
                    csv[k], civ[k], t0s, t1s, t2s, t0i, t1i, t2i)
            return (t0s, t1s, t2s, t0i, t1i, t2i)

        z = jnp.int32(0)
        ng = jnp.float32(NEG)
        t0s, t1s, t2s, t0i, t1i, t2i = lax.fori_loop(
            0, NW, m_body, (ng, ng, ng, z, z, z))

        cnt = pcv[0, pl.ds(0, 16)]
        cntc = jnp.maximum(cnt, jnp.float32(1.0))
        one = jnp.full((16,), 1.0, jnp.float32)
        v0 = jnp.where(cnt > 0.5, one, zeros16)
        v1 = jnp.where(cnt > 1.5, one, zeros16)
        v2 = jnp.where(cnt > 2.5, one, zeros16)

        gi_v[...] = jnp.where(iota == 0, t0i,
                              jnp.where(iota == 1, t1i,
                                        jnp.where(iota == 2, t2i, z)))
        pltpu.async_copy(x_hbm.at[gi_v], grows_v, sem_g).wait()

        def col_body(c4, _):
            bases = [pl.multiple_of(c4 * 64 + u * 16, 16) for u in range(4)]
            for u in range(4):
                base = bases[u]
                sl = pl.ds(base, 16)
                sv = psv[0, sl]
                orow_v[pl.ds(base, 16)] = sv / cntc
                orow_v[pl.ds(D + base, 16)] = sv
                orow_v[pl.ds(2 * D + base, 16)] = grows_v[0, sl] * v0
                orow_v[pl.ds(3 * D + base, 16)] = grows_v[1, sl] * v1
                orow_v[pl.ds(4 * D + base, 16)] = grows_v[2, sl] * v2
            return 0

        lax.fori_loop(0, D // 64, col_body, 0)
        pltpu.async_copy(orow_v, out_hbm.at[seg], sem_out)

    seg_a = wid * 2
    seg_b = seg_a + 1
    for c in in_copies(seg_a, cs0, ci0, ps0, pc0, sem_in0):
        c.start()
    for c in in_copies(seg_b, cs1, ci1, ps1, pc1, sem_in1):
        c.start()
    do_seg(seg_a, cs0, ci0, ps0, pc0, orow0, sem_in0)
    do_seg(seg_b, cs1, ci1, ps1, pc1, orow1, sem_in1)
    pltpu.make_async_copy(orow0, out_hbm.at[seg_a], sem_out).wait()
    pltpu.make_async_copy(orow1, out_hbm.at[seg_b], sem_out).wait()


def kernel(x, batch):
    bat = batch.astype(jnp.int32)
    psum, pcnt, scores = _ktc(x, bat)
    cs, ci = _ksc1(scores, bat)
    return _ksc2(x, psum, pcnt, cs, ci)
